# baseline (speedup 1.0000x reference)
import functools

import jax
import jax.numpy as jnp
from jax import lax
from jax.experimental import pallas as pl
from jax.experimental.pallas import tpu as pltpu
from jax.experimental.pallas import tpu_sc as plsc

NC = 2
NS = 16
CH = 128
NBUF = 4


@functools.lru_cache(maxsize=None)
def _make_agg(n_pad, h, nchunk, with_deg):
    hc = h // NC
    rows_pt = n_pad // NS
    ndrain = rows_pt // CH
    hcb = hc // 16

    mesh = plsc.VectorSubcoreMesh(
        core_axis_name="c", subcore_axis_name="s",
        num_cores=NC, num_subcores=NS)

    outs = [jax.ShapeDtypeStruct((NC, n_pad, hc), jnp.float32)]
    if with_deg:
        outs.append(jax.ShapeDtypeStruct((n_pad,), jnp.float32))

    scratch = [
        pltpu.VMEM((nchunk, CH), jnp.int32),
        pltpu.VMEM((nchunk, CH), jnp.int32),
        pltpu.VMEM((CH,), jnp.float32),
        pltpu.VMEM((rows_pt,), jnp.float32),
        pltpu.VMEM_SHARED((n_pad, hc), jnp.float32),
        pltpu.VMEM_SHARED((n_pad,), jnp.float32),
    ] + [pltpu.VMEM((CH, hc), jnp.float32) for _ in range(NBUF)] \
      + [pltpu.SemaphoreType.DMA for _ in range(2 * NBUF + 1)]

    def body(p_hbm, src_hbm, dst_hbm, *rest):
        if with_deg:
            out_hbm, invd_hbm = rest[0], rest[1]
            scr = rest[2:]
        else:
            out_hbm = rest[0]
            scr = rest[1:]
        srcv, dstv, onesv, dbuf, acc_sh, deg_sh = scr[:6]
        bufs = list(scr[6:6 + NBUF])
        gsems = list(scr[6 + NBUF:6 + 2 * NBUF])
        ssems = list(scr[6 + 2 * NBUF:6 + 3 * NBUF])
        dsem = scr[6 + 3 * NBUF]
        buf = bufs[0]

        c = lax.axis_index("c")
        s = lax.axis_index("s")
        base = s * rows_pt
        ptab = p_hbm.at[c]

        zero16 = jnp.zeros((16,), jnp.float32)

        def zfill(r, carry):
            for cb in range(hcb):
                buf[r, pl.ds(cb * 16, 16)] = zero16
            return carry
        lax.fori_loop(0, CH, zfill, 0)

        pltpu.sync_copy(src_hbm.at[s], srcv)
        pltpu.sync_copy(dst_hbm.at[s], dstv)
        if with_deg:
            one16 = jnp.ones((16,), jnp.float32)

            def ofill(i, carry):
                onesv[pl.ds(i * 16, 16)] = one16
                return carry
            lax.fori_loop(0, CH // 16, ofill, 0)

        def zcopy(i, carry):
            pltpu.sync_copy(buf, acc_sh.at[pl.ds(base + i * CH, CH)])
            return carry
        lax.fori_loop(0, ndrain, zcopy, 0)
        if with_deg:
            def zdeg(i, carry):
                pltpu.sync_copy(buf.at[0], deg_sh.at[pl.ds(base + i * hc, hc)])
                return carry
            lax.fori_loop(0, rows_pt // hc, zdeg, 0)
        plsc.subcore_barrier()

        LEAD = NBUF // 2
        for b in range(LEAD):
            pltpu.async_copy(ptab.at[srcv.at[b]], bufs[b], gsems[b])

        def group(g, carry):
            for b in range(NBUF):
                j = g * NBUF + b
                pltpu.make_async_copy(
                    ptab.at[srcv.at[j]], bufs[b], gsems[b]).wait()
                pltpu.async_copy(bufs[b], acc_sh.at[dstv.at[j]], ssems[b],
                                 add=True)
                if with_deg:
                    @pl.when(c == 0)
                    def _():
                        pltpu.async_copy(onesv, deg_sh.at[dstv.at[j]],
                                         dsem, add=True)
                nj = j + LEAD
                bn = (b + LEAD) % NBUF

                @pl.when(nj < nchunk)
                def _():
                    @pl.when(j >= LEAD)
                    def _():
                        pltpu.make_async_copy(
                            bufs[bn], acc_sh.at[dstv.at[j - LEAD]],
                            ssems[bn]).wait()
                    pltpu.async_copy(ptab.at[srcv.at[nj]], bufs[bn],
                                     gsems[bn])
            return carry
        lax.fori_loop(0, nchunk // NBUF, group, 0)
        for m in range(nchunk - 2 * LEAD, nchunk):
            pltpu.make_async_copy(
                bufs[m % NBUF], acc_sh.at[dstv.at[m]], ssems[m % NBUF]).wait()
        if with_deg:
            @pl.when(c == 0)
            def _():
                def ddrain(j, carry):
                    pltpu.make_async_copy(
                        onesv, deg_sh.at[dstv.at[j]], dsem).wait()
                    return carry
                lax.fori_loop(0, nchunk, ddrain, 0)
        plsc.subcore_barrier()

        if with_deg:
            pltpu.sync_copy(deg_sh.at[pl.ds(base, rows_pt)], dbuf)

            def iv(i, carry):
                d = dbuf[pl.ds(i * 16, 16)]
                dbuf[pl.ds(i * 16, 16)] = 1.0 / jnp.maximum(d, 1.0)
                return carry
            lax.fori_loop(0, rows_pt // 16, iv, 0)

            @pl.when(c == 0)
            def _():
                pltpu.sync_copy(dbuf, invd_hbm.at[pl.ds(base, rows_pt)])

        def dr(i, carry):
            pltpu.sync_copy(acc_sh.at[pl.ds(base + i * CH, CH)], buf)
            pltpu.sync_copy(buf, out_hbm.at[c].at[pl.ds(base + i * CH, CH)])
            return carry
        lax.fori_loop(0, ndrain, dr, 0)

    return pl.kernel(body, out_type=tuple(outs), mesh=mesh,
                     scratch_types=tuple(scratch),
                     compiler_params=pltpu.CompilerParams(
                         use_tc_tiling_on_sc=False))


def _mm2_body(x_ref, wn_ref, ws_ref, b_ref, p_ref, q_ref):
    x = x_ref[...]
    p = jnp.dot(x, wn_ref[...], preferred_element_type=jnp.float32)
    hc = p.shape[1] // NC
    p_ref[0] = p[:, :hc]
    p_ref[1] = p[:, hc:]
    q_ref[...] = (jnp.dot(x, ws_ref[...], preferred_element_type=jnp.float32)
                  + b_ref[...])


def _sp_concat(sp_ref, invd_ref):
    return (jnp.concatenate([sp_ref[0], sp_ref[1]], axis=-1)
            * invd_ref[...])


def _layer_body(q_ref, sp_ref, invd_ref, wn_ref, ws_ref, b_ref,
                p_ref, q2_ref):
    sm = _sp_concat(sp_ref, invd_ref)
    hcur = jnp.maximum(q_ref[...] + sm, 0.0)
    p = jnp.dot(hcur, wn_ref[...], preferred_element_type=jnp.float32)
    hc = p.shape[1] // NC
    p_ref[0] = p[:, :hc]
    p_ref[1] = p[:, hc:]
    q2_ref[...] = (jnp.dot(hcur, ws_ref[...],
                           preferred_element_type=jnp.float32) + b_ref[...])


def _make_final_body(n_real, rblk):
    def _final_body(q_ref, sp_ref, invd_ref, out_ref):
        i = pl.program_id(0)
        sm = _sp_concat(sp_ref, invd_ref)
        h2 = jnp.maximum(q_ref[...] + sm, 0.0)
        rows = i * rblk + lax.broadcasted_iota(jnp.int32, (rblk, 1), 0)
        h2 = jnp.where(rows < n_real, h2, 0.0)
        part = jnp.sum(h2, axis=0, keepdims=True) * (1.0 / n_real)

        @pl.when(i == 0)
        def _():
            out_ref[...] = jnp.zeros_like(out_ref)
        out_ref[...] += part
    return _final_body


def _tc1(feat_p, wn, ws, b, rblk):
    npad, d = feat_p.shape
    h = wn.shape[1]
    hc = h // NC
    return pl.pallas_call(
        _mm2_body,
        grid=(npad // rblk,),
        in_specs=[pl.BlockSpec((rblk, d), lambda i: (i, 0)),
                  pl.BlockSpec((d, h), lambda i: (0, 0)),
                  pl.BlockSpec((d, h), lambda i: (0, 0)),
                  pl.BlockSpec((1, h), lambda i: (0, 0))],
        out_specs=[pl.BlockSpec((NC, rblk, hc), lambda i: (0, i, 0)),
                   pl.BlockSpec((rblk, h), lambda i: (i, 0))],
        out_shape=[jax.ShapeDtypeStruct((NC, npad, hc), jnp.float32),
                   jax.ShapeDtypeStruct((npad, h), jnp.float32)],
    )(feat_p, wn, ws, b)


def _tc2(q, sp, invd, wn, ws, b, rblk):
    npad, h = q.shape
    hc = h // NC
    return pl.pallas_call(
        _layer_body,
        grid=(npad // rblk,),
        in_specs=[pl.BlockSpec((rblk, h), lambda i: (i, 0)),
                  pl.BlockSpec((NC, rblk, hc), lambda i: (0, i, 0)),
                  pl.BlockSpec((rblk, 1), lambda i: (i, 0)),
                  pl.BlockSpec((h, h), lambda i: (0, 0)),
                  pl.BlockSpec((h, h), lambda i: (0, 0)),
                  pl.BlockSpec((1, h), lambda i: (0, 0))],
        out_specs=[pl.BlockSpec((NC, rblk, hc), lambda i: (0, i, 0)),
                   pl.BlockSpec((rblk, h), lambda i: (i, 0))],
        out_shape=[jax.ShapeDtypeStruct((NC, npad, hc), jnp.float32),
                   jax.ShapeDtypeStruct((npad, h), jnp.float32)],
    )(q, sp, invd, wn, ws, b)


def _tc3(q, sp, invd, n_real, rblk):
    npad, h = q.shape
    hc = h // NC
    return pl.pallas_call(
        _make_final_body(n_real, rblk),
        grid=(npad // rblk,),
        in_specs=[pl.BlockSpec((rblk, h), lambda i: (i, 0)),
                  pl.BlockSpec((NC, rblk, hc), lambda i: (0, i, 0)),
                  pl.BlockSpec((rblk, 1), lambda i: (i, 0))],
        out_specs=pl.BlockSpec((1, h), lambda i: (0, 0)),
        out_shape=jax.ShapeDtypeStruct((1, h), jnp.float32),
    )(q, sp, invd)


def kernel(feat, edge_index, W_self1, W_neigh1, b1, W_self2, W_neigh2, b2):
    n, d = feat.shape
    e = edge_index.shape[1]
    h = W_self1.shape[0]
    rblk = 1280

    npad = -(-n // (NS * CH)) * NS * CH
    nchunk = -(-e // (NS * CH))
    nchunk = -(-nchunk // NBUF) * NBUF
    e_pad = NS * nchunk * CH
    pad = e_pad - e
    prows = npad - n

    src = edge_index[0].astype(jnp.int32)
    dst = edge_index[1].astype(jnp.int32)
    if pad:
        ar = jnp.arange(pad, dtype=jnp.int32)
        src = jnp.concatenate([src, ar % n])
        dst = jnp.concatenate([dst, n + ar % prows])
    src3 = src.reshape(NS, nchunk, CH)
    dst3 = dst.reshape(NS, nchunk, CH)

    feat_p = feat
    if npad != n:
        feat_p = jnp.concatenate(
            [feat, jnp.zeros((npad - n, d), feat.dtype)])

    wn1, ws1 = W_neigh1.T, W_self1.T
    wn2, ws2 = W_neigh2.T, W_self2.T
    b1r, b2r = b1.reshape(1, h), b2.reshape(1, h)

    p1, q1 = _tc1(feat_p, wn1, ws1, b1r, rblk)
    s1, invd = _make_agg(npad, h, nchunk, True)(p1, src3, dst3)
    invd2 = invd.reshape(npad, 1)
    p2, q2 = _tc2(q1, s1, invd2, wn2, ws2, b2r, rblk)
    (s2,) = _make_agg(npad, h, nchunk, False)(p2, src3, dst3)
    out = _tc3(q2, s2, invd2, n, rblk)
    return out.reshape(h)


# speedup vs baseline: 1.1262x; 1.1262x over previous
import functools

import jax
import jax.numpy as jnp
from jax import lax
from jax.experimental import pallas as pl
from jax.experimental.pallas import tpu as pltpu
from jax.experimental.pallas import tpu_sc as plsc

NC = 2
NS = 16
CH = 128
NBUF = 5
LEAD = 3


@functools.lru_cache(maxsize=None)
def _make_agg(n_pad, h, nchunk, with_deg):
    hc = h // NC
    rows_pt = n_pad // NS
    ndrain = rows_pt // CH
    hcb = hc // 16

    mesh = plsc.VectorSubcoreMesh(
        core_axis_name="c", subcore_axis_name="s",
        num_cores=NC, num_subcores=NS)

    outs = [jax.ShapeDtypeStruct((NC, n_pad, hc), jnp.float32)]
    if with_deg:
        outs.append(jax.ShapeDtypeStruct((n_pad,), jnp.float32))

    scratch = [
        pltpu.VMEM((nchunk, CH), jnp.int32),
        pltpu.VMEM((nchunk, CH), jnp.int32),
        pltpu.VMEM((CH,), jnp.float32),
        pltpu.VMEM((rows_pt,), jnp.float32),
        pltpu.VMEM_SHARED((n_pad, hc), jnp.float32),
        pltpu.VMEM_SHARED((n_pad,), jnp.float32),
    ] + [pltpu.VMEM((CH, hc), jnp.float32) for _ in range(NBUF)] \
      + [pltpu.SemaphoreType.DMA for _ in range(2 * NBUF + 1)]

    def body(p_hbm, src_hbm, dst_hbm, *rest):
        if with_deg:
            out_hbm, invd_hbm = rest[0], rest[1]
            scr = rest[2:]
        else:
            out_hbm = rest[0]
            scr = rest[1:]
        srcv, dstv, onesv, dbuf, acc_sh, deg_sh = scr[:6]
        bufs = list(scr[6:6 + NBUF])
        gsems = list(scr[6 + NBUF:6 + 2 * NBUF])
        ssems = list(scr[6 + 2 * NBUF:6 + 3 * NBUF])
        dsem = scr[6 + 3 * NBUF]
        buf = bufs[0]

        c = lax.axis_index("c")
        s = lax.axis_index("s")
        base = s * rows_pt
        ptab = p_hbm.at[c]

        zero16 = jnp.zeros((16,), jnp.float32)

        def zfill(r, carry):
            for cb in range(hcb):
                buf[r, pl.ds(cb * 16, 16)] = zero16
            return carry
        lax.fori_loop(0, CH, zfill, 0)

        pltpu.sync_copy(src_hbm.at[s], srcv)
        pltpu.sync_copy(dst_hbm.at[s], dstv)
        if with_deg:
            one16 = jnp.ones((16,), jnp.float32)

            def ofill(i, carry):
                onesv[pl.ds(i * 16, 16)] = one16
                return carry
            lax.fori_loop(0, CH // 16, ofill, 0)

        def zcopy(i, carry):
            pltpu.sync_copy(buf, acc_sh.at[pl.ds(base + i * CH, CH)])
            return carry
        lax.fori_loop(0, ndrain, zcopy, 0)
        if with_deg:
            def zdeg(i, carry):
                pltpu.sync_copy(buf.at[0], deg_sh.at[pl.ds(base + i * hc, hc)])
                return carry
            lax.fori_loop(0, rows_pt // hc, zdeg, 0)
        plsc.subcore_barrier()

        for b in range(LEAD):
            pltpu.async_copy(ptab.at[srcv.at[b]], bufs[b], gsems[b])

        def group(g, carry):
            for b in range(NBUF):
                j = g * NBUF + b
                pltpu.make_async_copy(
                    ptab.at[srcv.at[j]], bufs[b], gsems[b]).wait()
                pltpu.async_copy(bufs[b], acc_sh.at[dstv.at[j]], ssems[b],
                                 add=True)
                if with_deg:
                    @pl.when(c == 0)
                    def _():
                        pltpu.async_copy(onesv, deg_sh.at[dstv.at[j]],
                                         dsem, add=True)
                nj = j + LEAD
                bn = (b + LEAD) % NBUF
                slack = NBUF - LEAD

                @pl.when(nj < nchunk)
                def _():
                    @pl.when(j >= slack)
                    def _():
                        pltpu.make_async_copy(
                            bufs[bn], acc_sh.at[dstv.at[j - slack]],
                            ssems[bn]).wait()
                    pltpu.async_copy(ptab.at[srcv.at[nj]], bufs[bn],
                                     gsems[bn])
            return carry
        lax.fori_loop(0, nchunk // NBUF, group, 0)
        for m in range(nchunk - NBUF, nchunk):
            pltpu.make_async_copy(
                bufs[m % NBUF], acc_sh.at[dstv.at[m]], ssems[m % NBUF]).wait()
        if with_deg:
            @pl.when(c == 0)
            def _():
                def ddrain(j, carry):
                    pltpu.make_async_copy(
                        onesv, deg_sh.at[dstv.at[j]], dsem).wait()
                    return carry
                lax.fori_loop(0, nchunk, ddrain, 0)
        plsc.subcore_barrier()

        if with_deg:
            pltpu.sync_copy(deg_sh.at[pl.ds(base, rows_pt)], dbuf)

            def iv(i, carry):
                d = dbuf[pl.ds(i * 16, 16)]
                dbuf[pl.ds(i * 16, 16)] = 1.0 / jnp.maximum(d, 1.0)
                return carry
            lax.fori_loop(0, rows_pt // 16, iv, 0)

            @pl.when(c == 0)
            def _():
                pltpu.sync_copy(dbuf, invd_hbm.at[pl.ds(base, rows_pt)])

        def dr(i, carry):
            pltpu.sync_copy(acc_sh.at[pl.ds(base + i * CH, CH)], buf)
            pltpu.sync_copy(buf, out_hbm.at[c].at[pl.ds(base + i * CH, CH)])
            return carry
        lax.fori_loop(0, ndrain, dr, 0)

    return pl.kernel(body, out_type=tuple(outs), mesh=mesh,
                     scratch_types=tuple(scratch),
                     compiler_params=pltpu.CompilerParams(
                         use_tc_tiling_on_sc=False))


def _mm2_body(x_ref, wn_ref, ws_ref, b_ref, p_ref, q_ref):
    x = x_ref[...]
    p = jnp.dot(x, wn_ref[...], preferred_element_type=jnp.float32)
    hc = p.shape[1] // NC
    p_ref[0] = p[:, :hc]
    p_ref[1] = p[:, hc:]
    q_ref[...] = (jnp.dot(x, ws_ref[...], preferred_element_type=jnp.float32)
                  + b_ref[...])


def _sp_concat(sp_ref, invd_ref):
    return (jnp.concatenate([sp_ref[0], sp_ref[1]], axis=-1)
            * invd_ref[...])


def _layer_body(q_ref, sp_ref, invd_ref, wn_ref, ws_ref, b_ref,
                p_ref, q2_ref):
    sm = _sp_concat(sp_ref, invd_ref)
    hcur = jnp.maximum(q_ref[...] + sm, 0.0)
    p = jnp.dot(hcur, wn_ref[...], preferred_element_type=jnp.float32)
    hc = p.shape[1] // NC
    p_ref[0] = p[:, :hc]
    p_ref[1] = p[:, hc:]
    q2_ref[...] = (jnp.dot(hcur, ws_ref[...],
                           preferred_element_type=jnp.float32) + b_ref[...])


def _make_final_body(n_real, rblk):
    def _final_body(q_ref, sp_ref, invd_ref, out_ref):
        i = pl.program_id(0)
        sm = _sp_concat(sp_ref, invd_ref)
        h2 = jnp.maximum(q_ref[...] + sm, 0.0)
        rows = i * rblk + lax.broadcasted_iota(jnp.int32, (rblk, 1), 0)
        h2 = jnp.where(rows < n_real, h2, 0.0)
        part = jnp.sum(h2, axis=0, keepdims=True) * (1.0 / n_real)

        @pl.when(i == 0)
        def _():
            out_ref[...] = jnp.zeros_like(out_ref)
        out_ref[...] += part
    return _final_body


def _tc1(feat_p, wn, ws, b, rblk):
    npad, d = feat_p.shape
    h = wn.shape[1]
    hc = h // NC
    return pl.pallas_call(
        _mm2_body,
        grid=(npad // rblk,),
        in_specs=[pl.BlockSpec((rblk, d), lambda i: (i, 0)),
                  pl.BlockSpec((d, h), lambda i: (0, 0)),
                  pl.BlockSpec((d, h), lambda i: (0, 0)),
                  pl.BlockSpec((1, h), lambda i: (0, 0))],
        out_specs=[pl.BlockSpec((NC, rblk, hc), lambda i: (0, i, 0)),
                   pl.BlockSpec((rblk, h), lambda i: (i, 0))],
        out_shape=[jax.ShapeDtypeStruct((NC, npad, hc), jnp.float32),
                   jax.ShapeDtypeStruct((npad, h), jnp.float32)],
    )(feat_p, wn, ws, b)


def _tc2(q, sp, invd, wn, ws, b, rblk):
    npad, h = q.shape
    hc = h // NC
    return pl.pallas_call(
        _layer_body,
        grid=(npad // rblk,),
        in_specs=[pl.BlockSpec((rblk, h), lambda i: (i, 0)),
                  pl.BlockSpec((NC, rblk, hc), lambda i: (0, i, 0)),
                  pl.BlockSpec((rblk, 1), lambda i: (i, 0)),
                  pl.BlockSpec((h, h), lambda i: (0, 0)),
                  pl.BlockSpec((h, h), lambda i: (0, 0)),
                  pl.BlockSpec((1, h), lambda i: (0, 0))],
        out_specs=[pl.BlockSpec((NC, rblk, hc), lambda i: (0, i, 0)),
                   pl.BlockSpec((rblk, h), lambda i: (i, 0))],
        out_shape=[jax.ShapeDtypeStruct((NC, npad, hc), jnp.float32),
                   jax.ShapeDtypeStruct((npad, h), jnp.float32)],
    )(q, sp, invd, wn, ws, b)


def _tc3(q, sp, invd, n_real, rblk):
    npad, h = q.shape
    hc = h // NC
    return pl.pallas_call(
        _make_final_body(n_real, rblk),
        grid=(npad // rblk,),
        in_specs=[pl.BlockSpec((rblk, h), lambda i: (i, 0)),
                  pl.BlockSpec((NC, rblk, hc), lambda i: (0, i, 0)),
                  pl.BlockSpec((rblk, 1), lambda i: (i, 0))],
        out_specs=pl.BlockSpec((1, h), lambda i: (0, 0)),
        out_shape=jax.ShapeDtypeStruct((1, h), jnp.float32),
    )(q, sp, invd)


def kernel(feat, edge_index, W_self1, W_neigh1, b1, W_self2, W_neigh2, b2):
    n, d = feat.shape
    e = edge_index.shape[1]
    h = W_self1.shape[0]
    rblk = 1280

    npad = -(-n // (NS * CH)) * NS * CH
    nchunk = -(-e // (NS * CH))
    nchunk = -(-nchunk // NBUF) * NBUF
    e_pad = NS * nchunk * CH
    pad = e_pad - e
    prows = npad - n

    src = edge_index[0].astype(jnp.int32)
    dst = edge_index[1].astype(jnp.int32)
    if pad:
        ar = jnp.arange(pad, dtype=jnp.int32)
        src = jnp.concatenate([src, ar % n])
        dst = jnp.concatenate([dst, n + ar % prows])
    src3 = src.reshape(NS, nchunk, CH)
    dst3 = dst.reshape(NS, nchunk, CH)

    feat_p = feat
    if npad != n:
        feat_p = jnp.concatenate(
            [feat, jnp.zeros((npad - n, d), feat.dtype)])

    wn1, ws1 = W_neigh1.T, W_self1.T
    wn2, ws2 = W_neigh2.T, W_self2.T
    b1r, b2r = b1.reshape(1, h), b2.reshape(1, h)

    p1, q1 = _tc1(feat_p, wn1, ws1, b1r, rblk)
    s1, invd = _make_agg(npad, h, nchunk, True)(p1, src3, dst3)
    invd2 = invd.reshape(npad, 1)
    p2, q2 = _tc2(q1, s1, invd2, wn2, ws2, b2r, rblk)
    (s2,) = _make_agg(npad, h, nchunk, False)(p2, src3, dst3)
    out = _tc3(q2, s2, invd2, n, rblk)
    return out.reshape(h)


# speedup vs baseline: 1.1687x; 1.0377x over previous
import functools

import jax
import jax.numpy as jnp
from jax import lax
from jax.experimental import pallas as pl
from jax.experimental.pallas import tpu as pltpu
from jax.experimental.pallas import tpu_sc as plsc

NC = 2
NS = 16
CH = 128
NBUF = 5
LEAD = 3


@functools.lru_cache(maxsize=None)
def _make_agg(n_pad, h, nchunk, with_deg):
    hc = h // NC
    rows_pt = n_pad // NS
    ndrain = rows_pt // CH
    hcb = hc // 16

    mesh = plsc.VectorSubcoreMesh(
        core_axis_name="c", subcore_axis_name="s",
        num_cores=NC, num_subcores=NS)

    outs = [jax.ShapeDtypeStruct((NC, n_pad, hc), jnp.float32)]
    if with_deg:
        outs.append(jax.ShapeDtypeStruct((n_pad,), jnp.float32))

    scratch = [
        pltpu.VMEM((nchunk, CH), jnp.int32),
        pltpu.VMEM((nchunk, CH), jnp.int32),
        pltpu.VMEM((CH,), jnp.float32),
        pltpu.VMEM((rows_pt,), jnp.float32),
        pltpu.VMEM_SHARED((n_pad, hc), jnp.float32),
        pltpu.VMEM_SHARED((n_pad,), jnp.float32),
    ] + [pltpu.VMEM((CH, hc), jnp.float32) for _ in range(NBUF)] \
      + [pltpu.SemaphoreType.DMA for _ in range(2 * NBUF + 1)]

    def body(p_hbm, src_hbm, dst_hbm, *rest):
        if with_deg:
            out_hbm, invd_hbm = rest[0], rest[1]
            scr = rest[2:]
        else:
            out_hbm = rest[0]
            scr = rest[1:]
        srcv, dstv, onesv, dbuf, acc_sh, deg_sh = scr[:6]
        bufs = list(scr[6:6 + NBUF])
        gsems = list(scr[6 + NBUF:6 + 2 * NBUF])
        ssems = list(scr[6 + 2 * NBUF:6 + 3 * NBUF])
        dsem = scr[6 + 3 * NBUF]
        buf = bufs[0]

        c = lax.axis_index("c")
        s = lax.axis_index("s")
        base = s * rows_pt
        ptab = p_hbm.at[c]

        zero16 = jnp.zeros((16,), jnp.float32)

        def zfill(r, carry):
            for cb in range(hcb):
                buf[r, pl.ds(cb * 16, 16)] = zero16
            return carry
        lax.fori_loop(0, CH, zfill, 0)

        pltpu.sync_copy(src_hbm.at[s], srcv)
        pltpu.sync_copy(dst_hbm.at[s], dstv)
        if with_deg:
            one16 = jnp.ones((16,), jnp.float32)

            def ofill(i, carry):
                onesv[pl.ds(i * 16, 16)] = one16
                return carry
            lax.fori_loop(0, CH // 16, ofill, 0)

        def zcopy(i, carry):
            pltpu.sync_copy(buf, acc_sh.at[pl.ds(base + i * CH, CH)])
            return carry
        lax.fori_loop(0, ndrain, zcopy, 0)
        if with_deg:
            def zdeg(i, carry):
                pltpu.sync_copy(buf.at[0], deg_sh.at[pl.ds(base + i * hc, hc)])
                return carry
            lax.fori_loop(0, rows_pt // hc, zdeg, 0)
        plsc.subcore_barrier()

        for b in range(NBUF):
            pltpu.async_copy(ptab.at[srcv.at[b]], bufs[b], gsems[b])

        def group(g, carry):
            for b in range(NBUF):
                j = g * NBUF + b
                pltpu.make_async_copy(
                    ptab.at[srcv.at[j]], bufs[b], gsems[b]).wait()
                pltpu.sync_copy(bufs[b], acc_sh.at[dstv.at[j]], add=True)
                if with_deg:
                    @pl.when(c == 0)
                    def _():
                        pltpu.async_copy(onesv, deg_sh.at[dstv.at[j]],
                                         dsem, add=True)
                nj = j + NBUF

                @pl.when(nj < nchunk)
                def _():
                    pltpu.async_copy(ptab.at[srcv.at[nj]], bufs[b], gsems[b])
            return carry
        lax.fori_loop(0, nchunk // NBUF, group, 0)
        if with_deg:
            @pl.when(c == 0)
            def _():
                def ddrain(j, carry):
                    pltpu.make_async_copy(
                        onesv, deg_sh.at[dstv.at[j]], dsem).wait()
                    return carry
                lax.fori_loop(0, nchunk, ddrain, 0)
        plsc.subcore_barrier()

        if with_deg:
            pltpu.sync_copy(deg_sh.at[pl.ds(base, rows_pt)], dbuf)

            def iv(i, carry):
                d = dbuf[pl.ds(i * 16, 16)]
                dbuf[pl.ds(i * 16, 16)] = 1.0 / jnp.maximum(d, 1.0)
                return carry
            lax.fori_loop(0, rows_pt // 16, iv, 0)

            @pl.when(c == 0)
            def _():
                pltpu.sync_copy(dbuf, invd_hbm.at[pl.ds(base, rows_pt)])

        def dr(i, carry):
            pltpu.sync_copy(acc_sh.at[pl.ds(base + i * CH, CH)], buf)
            pltpu.sync_copy(buf, out_hbm.at[c].at[pl.ds(base + i * CH, CH)])
            return carry
        lax.fori_loop(0, ndrain, dr, 0)

    return pl.kernel(body, out_type=tuple(outs), mesh=mesh,
                     scratch_types=tuple(scratch),
                     compiler_params=pltpu.CompilerParams(
                         use_tc_tiling_on_sc=False))


def _mm2_body(x_ref, wn_ref, ws_ref, b_ref, p_ref, q_ref):
    x = x_ref[...]
    p = jnp.dot(x, wn_ref[...], preferred_element_type=jnp.float32)
    hc = p.shape[1] // NC
    p_ref[0] = p[:, :hc]
    p_ref[1] = p[:, hc:]
    q_ref[...] = (jnp.dot(x, ws_ref[...], preferred_element_type=jnp.float32)
                  + b_ref[...])


def _sp_concat(sp_ref, invd_ref):
    return (jnp.concatenate([sp_ref[0], sp_ref[1]], axis=-1)
            * invd_ref[...])


def _layer_body(q_ref, sp_ref, invd_ref, wn_ref, ws_ref, b_ref,
                p_ref, q2_ref):
    sm = _sp_concat(sp_ref, invd_ref)
    hcur = jnp.maximum(q_ref[...] + sm, 0.0)
    p = jnp.dot(hcur, wn_ref[...], preferred_element_type=jnp.float32)
    hc = p.shape[1] // NC
    p_ref[0] = p[:, :hc]
    p_ref[1] = p[:, hc:]
    q2_ref[...] = (jnp.dot(hcur, ws_ref[...],
                           preferred_element_type=jnp.float32) + b_ref[...])


def _make_final_body(n_real, rblk):
    def _final_body(q_ref, sp_ref, invd_ref, out_ref):
        i = pl.program_id(0)
        sm = _sp_concat(sp_ref, invd_ref)
        h2 = jnp.maximum(q_ref[...] + sm, 0.0)
        rows = i * rblk + lax.broadcasted_iota(jnp.int32, (rblk, 1), 0)
        h2 = jnp.where(rows < n_real, h2, 0.0)
        part = jnp.sum(h2, axis=0, keepdims=True) * (1.0 / n_real)

        @pl.when(i == 0)
        def _():
            out_ref[...] = jnp.zeros_like(out_ref)
        out_ref[...] += part
    return _final_body


def _tc1(feat_p, wn, ws, b, rblk):
    npad, d = feat_p.shape
    h = wn.shape[1]
    hc = h // NC
    return pl.pallas_call(
        _mm2_body,
        grid=(npad // rblk,),
        in_specs=[pl.BlockSpec((rblk, d), lambda i: (i, 0)),
                  pl.BlockSpec((d, h), lambda i: (0, 0)),
                  pl.BlockSpec((d, h), lambda i: (0, 0)),
                  pl.BlockSpec((1, h), lambda i: (0, 0))],
        out_specs=[pl.BlockSpec((NC, rblk, hc), lambda i: (0, i, 0)),
                   pl.BlockSpec((rblk, h), lambda i: (i, 0))],
        out_shape=[jax.ShapeDtypeStruct((NC, npad, hc), jnp.float32),
                   jax.ShapeDtypeStruct((npad, h), jnp.float32)],
    )(feat_p, wn, ws, b)


def _tc2(q, sp, invd, wn, ws, b, rblk):
    npad, h = q.shape
    hc = h // NC
    return pl.pallas_call(
        _layer_body,
        grid=(npad // rblk,),
        in_specs=[pl.BlockSpec((rblk, h), lambda i: (i, 0)),
                  pl.BlockSpec((NC, rblk, hc), lambda i: (0, i, 0)),
                  pl.BlockSpec((rblk, 1), lambda i: (i, 0)),
                  pl.BlockSpec((h, h), lambda i: (0, 0)),
                  pl.BlockSpec((h, h), lambda i: (0, 0)),
                  pl.BlockSpec((1, h), lambda i: (0, 0))],
        out_specs=[pl.BlockSpec((NC, rblk, hc), lambda i: (0, i, 0)),
                   pl.BlockSpec((rblk, h), lambda i: (i, 0))],
        out_shape=[jax.ShapeDtypeStruct((NC, npad, hc), jnp.float32),
                   jax.ShapeDtypeStruct((npad, h), jnp.float32)],
    )(q, sp, invd, wn, ws, b)


def _tc3(q, sp, invd, n_real, rblk):
    npad, h = q.shape
    hc = h // NC
    return pl.pallas_call(
        _make_final_body(n_real, rblk),
        grid=(npad // rblk,),
        in_specs=[pl.BlockSpec((rblk, h), lambda i: (i, 0)),
                  pl.BlockSpec((NC, rblk, hc), lambda i: (0, i, 0)),
                  pl.BlockSpec((rblk, 1), lambda i: (i, 0))],
        out_specs=pl.BlockSpec((1, h), lambda i: (0, 0)),
        out_shape=jax.ShapeDtypeStruct((1, h), jnp.float32),
    )(q, sp, invd)


def kernel(feat, edge_index, W_self1, W_neigh1, b1, W_self2, W_neigh2, b2):
    n, d = feat.shape
    e = edge_index.shape[1]
    h = W_self1.shape[0]
    rblk = 1280

    npad = -(-n // (NS * CH)) * NS * CH
    nchunk = -(-e // (NS * CH))
    nchunk = -(-nchunk // NBUF) * NBUF
    e_pad = NS * nchunk * CH
    pad = e_pad - e
    prows = npad - n

    src = edge_index[0].astype(jnp.int32)
    dst = edge_index[1].astype(jnp.int32)
    if pad:
        ar = jnp.arange(pad, dtype=jnp.int32)
        src = jnp.concatenate([src, ar % n])
        dst = jnp.concatenate([dst, n + ar % prows])
    src3 = src.reshape(NS, nchunk, CH)
    dst3 = dst.reshape(NS, nchunk, CH)

    feat_p = feat
    if npad != n:
        feat_p = jnp.concatenate(
            [feat, jnp.zeros((npad - n, d), feat.dtype)])

    wn1, ws1 = W_neigh1.T, W_self1.T
    wn2, ws2 = W_neigh2.T, W_self2.T
    b1r, b2r = b1.reshape(1, h), b2.reshape(1, h)

    p1, q1 = _tc1(feat_p, wn1, ws1, b1r, rblk)
    s1, invd = _make_agg(npad, h, nchunk, True)(p1, src3, dst3)
    invd2 = invd.reshape(npad, 1)
    p2, q2 = _tc2(q1, s1, invd2, wn2, ws2, b2r, rblk)
    (s2,) = _make_agg(npad, h, nchunk, False)(p2, src3, dst3)
    out = _tc3(q2, s2, invd2, n, rblk)
    return out.reshape(h)


# speedup vs baseline: 1.3023x; 1.1143x over previous
import functools

import jax
import jax.numpy as jnp
from jax import lax
from jax.experimental import pallas as pl
from jax.experimental.pallas import tpu as pltpu
from jax.experimental.pallas import tpu_sc as plsc

NC = 2
NS = 16
CH = 128
NBUF = 5
LEAD = 3


@functools.lru_cache(maxsize=None)
def _make_agg(n_pad, h, nchunk, with_deg):
    hc = h // NC
    rows_pt = n_pad // NS
    ndrain = rows_pt // CH
    hcb = hc // 16

    mesh = plsc.VectorSubcoreMesh(
        core_axis_name="c", subcore_axis_name="s",
        num_cores=NC, num_subcores=NS)

    outs = [jax.ShapeDtypeStruct((NC, n_pad, hc), jnp.float32)]
    if with_deg:
        outs.append(jax.ShapeDtypeStruct((n_pad,), jnp.float32))

    scratch = [
        pltpu.VMEM((nchunk, CH), jnp.int32),
        pltpu.VMEM((nchunk, CH), jnp.int32),
        pltpu.VMEM((CH,), jnp.float32),
        pltpu.VMEM((rows_pt,), jnp.float32),
        pltpu.VMEM_SHARED((n_pad, hc), jnp.float32),
        pltpu.VMEM_SHARED((n_pad,), jnp.float32),
    ] + [pltpu.VMEM((CH, hc), jnp.float32) for _ in range(NBUF)] \
      + [pltpu.SemaphoreType.DMA for _ in range(2 * NBUF + 1)]

    def body(p_hbm, src_hbm, dst_hbm, *rest):
        if with_deg:
            out_hbm, invd_hbm = rest[0], rest[1]
            scr = rest[2:]
        else:
            out_hbm = rest[0]
            scr = rest[1:]
        srcv, dstv, onesv, dbuf, acc_sh, deg_sh = scr[:6]
        bufs = list(scr[6:6 + NBUF])
        gsems = list(scr[6 + NBUF:6 + 2 * NBUF])
        ssems = list(scr[6 + 2 * NBUF:6 + 3 * NBUF])
        dsem = scr[6 + 3 * NBUF]
        buf = bufs[0]

        c = lax.axis_index("c")
        s = lax.axis_index("s")
        base = s * rows_pt
        ptab = p_hbm.at[c]

        zero16 = jnp.zeros((16,), jnp.float32)

        def zfill(r, carry):
            for cb in range(hcb):
                buf[r, pl.ds(cb * 16, 16)] = zero16
            return carry
        lax.fori_loop(0, CH, zfill, 0)

        pltpu.sync_copy(src_hbm.at[s], srcv)
        pltpu.sync_copy(dst_hbm.at[s], dstv)
        if with_deg:
            one16 = jnp.ones((16,), jnp.float32)

            def ofill(i, carry):
                onesv[pl.ds(i * 16, 16)] = one16
                return carry
            lax.fori_loop(0, CH // 16, ofill, 0)

        def zcopy(i, carry):
            pltpu.sync_copy(buf, acc_sh.at[pl.ds(base + i * CH, CH)])
            return carry
        lax.fori_loop(0, ndrain, zcopy, 0)
        if with_deg:
            def zdeg(i, carry):
                pltpu.sync_copy(buf.at[0], deg_sh.at[pl.ds(base + i * hc, hc)])
                return carry
            lax.fori_loop(0, rows_pt // hc, zdeg, 0)
        plsc.subcore_barrier()

        for b in range(NBUF):
            pltpu.async_copy(ptab.at[srcv.at[b]], bufs[b], gsems[b])

        def group(g, carry):
            for b in range(NBUF):
                j = g * NBUF + b
                pltpu.make_async_copy(
                    ptab.at[srcv.at[j]], bufs[b], gsems[b]).wait()
                pltpu.sync_copy(bufs[b], acc_sh.at[dstv.at[j]], add=True)
                if with_deg:
                    @pl.when(c == 0)
                    def _():
                        pltpu.async_copy(onesv, deg_sh.at[dstv.at[j]],
                                         dsem, add=True)
                nj = j + NBUF

                @pl.when(nj < nchunk)
                def _():
                    pltpu.async_copy(ptab.at[srcv.at[nj]], bufs[b], gsems[b])
            return carry
        lax.fori_loop(0, nchunk // NBUF, group, 0)
        if with_deg:
            @pl.when(c == 0)
            def _():
                def ddrain(j, carry):
                    pltpu.make_async_copy(
                        onesv, deg_sh.at[dstv.at[j]], dsem).wait()
                    return carry
                lax.fori_loop(0, nchunk, ddrain, 0)
        plsc.subcore_barrier()

        if with_deg:
            pltpu.sync_copy(deg_sh.at[pl.ds(base, rows_pt)], dbuf)

            def iv(i, carry):
                d = dbuf[pl.ds(i * 16, 16)]
                dbuf[pl.ds(i * 16, 16)] = 1.0 / jnp.maximum(d, 1.0)
                return carry
            lax.fori_loop(0, rows_pt // 16, iv, 0)

            @pl.when(c == 0)
            def _():
                pltpu.sync_copy(dbuf, invd_hbm.at[pl.ds(base, rows_pt)])

        def dr(i, carry):
            pltpu.sync_copy(acc_sh.at[pl.ds(base + i * CH, CH)], buf)
            pltpu.sync_copy(buf, out_hbm.at[c].at[pl.ds(base + i * CH, CH)])
            return carry
        lax.fori_loop(0, ndrain, dr, 0)

    return pl.kernel(body, out_type=tuple(outs), mesh=mesh,
                     scratch_types=tuple(scratch),
                     compiler_params=pltpu.CompilerParams(
                         use_tc_tiling_on_sc=False))


def _pack_p(p, p_ref):
    rb, h = p.shape
    hc = h // NC
    p3 = p.reshape(rb // 2, 2, h)
    for c in range(NC):
        p_ref[c] = jnp.concatenate(
            [p3[:, 0, c * hc:(c + 1) * hc], p3[:, 1, c * hc:(c + 1) * hc]],
            axis=1)


def _unpack_s(sp_ref, invd_ref):
    spp = sp_ref[...]
    _, rb2, h = spp.shape
    hc = h // 2
    a = jnp.concatenate([spp[0][:, :hc], spp[1][:, :hc]], axis=1)
    b = jnp.concatenate([spp[0][:, hc:], spp[1][:, hc:]], axis=1)
    sm = jnp.stack([a, b], axis=1).reshape(rb2 * 2, h)
    return sm * invd_ref[...]


def _mm2_body(x_ref, wn_ref, ws_ref, b_ref, p_ref, q_ref):
    x = x_ref[...]
    _pack_p(jnp.dot(x, wn_ref[...], preferred_element_type=jnp.float32),
            p_ref)
    q_ref[...] = (jnp.dot(x, ws_ref[...], preferred_element_type=jnp.float32)
                  + b_ref[...])


def _layer_body(q_ref, sp_ref, invd_ref, wn_ref, ws_ref, b_ref,
                p_ref, q2_ref):
    sm = _unpack_s(sp_ref, invd_ref)
    hcur = jnp.maximum(q_ref[...] + sm, 0.0)
    _pack_p(jnp.dot(hcur, wn_ref[...], preferred_element_type=jnp.float32),
            p_ref)
    q2_ref[...] = (jnp.dot(hcur, ws_ref[...],
                           preferred_element_type=jnp.float32) + b_ref[...])


def _make_final_body(n_real, rblk):
    def _final_body(q_ref, sp_ref, invd_ref, out_ref):
        i = pl.program_id(0)
        sm = _unpack_s(sp_ref, invd_ref)
        h2 = jnp.maximum(q_ref[...] + sm, 0.0)
        rows = i * rblk + lax.broadcasted_iota(jnp.int32, (rblk, 1), 0)
        h2 = jnp.where(rows < n_real, h2, 0.0)
        part = jnp.sum(h2, axis=0, keepdims=True) * (1.0 / n_real)

        @pl.when(i == 0)
        def _():
            out_ref[...] = jnp.zeros_like(out_ref)
        out_ref[...] += part
    return _final_body


def _tc1(feat_p, wn, ws, b, rblk):
    npad, d = feat_p.shape
    h = wn.shape[1]
    return pl.pallas_call(
        _mm2_body,
        grid=(npad // rblk,),
        in_specs=[pl.BlockSpec((rblk, d), lambda i: (i, 0)),
                  pl.BlockSpec((d, h), lambda i: (0, 0)),
                  pl.BlockSpec((d, h), lambda i: (0, 0)),
                  pl.BlockSpec((1, h), lambda i: (0, 0))],
        out_specs=[pl.BlockSpec((NC, rblk // 2, h), lambda i: (0, i, 0)),
                   pl.BlockSpec((rblk, h), lambda i: (i, 0))],
        out_shape=[jax.ShapeDtypeStruct((NC, npad // 2, h), jnp.float32),
                   jax.ShapeDtypeStruct((npad, h), jnp.float32)],
    )(feat_p, wn, ws, b)


def _tc2(q, sp, invd, wn, ws, b, rblk):
    npad, h = q.shape
    return pl.pallas_call(
        _layer_body,
        grid=(npad // rblk,),
        in_specs=[pl.BlockSpec((rblk, h), lambda i: (i, 0)),
                  pl.BlockSpec((NC, rblk // 2, h), lambda i: (0, i, 0)),
                  pl.BlockSpec((rblk, 1), lambda i: (i, 0)),
                  pl.BlockSpec((h, h), lambda i: (0, 0)),
                  pl.BlockSpec((h, h), lambda i: (0, 0)),
                  pl.BlockSpec((1, h), lambda i: (0, 0))],
        out_specs=[pl.BlockSpec((NC, rblk // 2, h), lambda i: (0, i, 0)),
                   pl.BlockSpec((rblk, h), lambda i: (i, 0))],
        out_shape=[jax.ShapeDtypeStruct((NC, npad // 2, h), jnp.float32),
                   jax.ShapeDtypeStruct((npad, h), jnp.float32)],
    )(q, sp, invd, wn, ws, b)


def _tc3(q, sp, invd, n_real, rblk):
    npad, h = q.shape
    return pl.pallas_call(
        _make_final_body(n_real, rblk),
        grid=(npad // rblk,),
        in_specs=[pl.BlockSpec((rblk, h), lambda i: (i, 0)),
                  pl.BlockSpec((NC, rblk // 2, h), lambda i: (0, i, 0)),
                  pl.BlockSpec((rblk, 1), lambda i: (i, 0))],
        out_specs=pl.BlockSpec((1, h), lambda i: (0, 0)),
        out_shape=jax.ShapeDtypeStruct((1, h), jnp.float32),
    )(q, sp, invd)


def kernel(feat, edge_index, W_self1, W_neigh1, b1, W_self2, W_neigh2, b2):
    n, d = feat.shape
    e = edge_index.shape[1]
    h = W_self1.shape[0]
    rblk = 1280

    npad = -(-n // (NS * CH)) * NS * CH
    nchunk = -(-e // (NS * CH))
    nchunk = -(-nchunk // NBUF) * NBUF
    e_pad = NS * nchunk * CH
    pad = e_pad - e
    prows = npad - n

    src = edge_index[0].astype(jnp.int32)
    dst = edge_index[1].astype(jnp.int32)
    if pad:
        ar = jnp.arange(pad, dtype=jnp.int32)
        src = jnp.concatenate([src, ar % n])
        dst = jnp.concatenate([dst, n + ar % prows])
    src3 = src.reshape(NS, nchunk, CH)
    dst3 = dst.reshape(NS, nchunk, CH)

    feat_p = feat
    if npad != n:
        feat_p = jnp.concatenate(
            [feat, jnp.zeros((npad - n, d), feat.dtype)])

    wn1, ws1 = W_neigh1.T, W_self1.T
    wn2, ws2 = W_neigh2.T, W_self2.T
    b1r, b2r = b1.reshape(1, h), b2.reshape(1, h)

    hc = h // NC
    agg1 = _make_agg(npad, h, nchunk, True)
    agg2 = _make_agg(npad, h, nchunk, False)

    p1, q1 = _tc1(feat_p, wn1, ws1, b1r, rblk)
    s1, invd = agg1(p1.reshape(NC, npad, hc), src3, dst3)
    invd2 = invd.reshape(npad, 1)
    p2, q2 = _tc2(q1, s1.reshape(NC, npad // 2, h), invd2, wn2, ws2, b2r,
                  rblk)
    (s2,) = agg2(p2.reshape(NC, npad, hc), src3, dst3)
    out = _tc3(q2, s2.reshape(NC, npad // 2, h), invd2, n, rblk)
    return out.reshape(h)


# speedup vs baseline: 1.3047x; 1.0018x over previous
import functools

import jax
import jax.numpy as jnp
from jax import lax
from jax.experimental import pallas as pl
from jax.experimental.pallas import tpu as pltpu
from jax.experimental.pallas import tpu_sc as plsc

NC = 2
NS = 16
CH = 128
NBUF = 5
LEAD = 3


@functools.lru_cache(maxsize=None)
def _make_agg(n_pad, h, nchunk, with_deg):
    hc = h // NC
    rows_pt = n_pad // NS
    ndrain = rows_pt // CH
    hcb = hc // 16

    mesh = plsc.VectorSubcoreMesh(
        core_axis_name="c", subcore_axis_name="s",
        num_cores=NC, num_subcores=NS)

    outs = [jax.ShapeDtypeStruct((NC, n_pad, hc), jnp.float32)]
    if with_deg:
        outs.append(jax.ShapeDtypeStruct((n_pad,), jnp.float32))

    scratch = [
        pltpu.VMEM((nchunk, CH), jnp.int32),
        pltpu.VMEM((nchunk, CH), jnp.int32),
        pltpu.VMEM((CH,), jnp.float32),
        pltpu.VMEM((rows_pt,), jnp.float32),
        pltpu.VMEM_SHARED((n_pad, hc), jnp.float32),
        pltpu.VMEM_SHARED((n_pad,), jnp.float32),
    ] + [pltpu.VMEM((CH, hc), jnp.float32) for _ in range(NBUF)] \
      + [pltpu.SemaphoreType.DMA for _ in range(2 * NBUF + 1)]

    def body(p_hbm, src_hbm, dst_hbm, *rest):
        if with_deg:
            out_hbm, invd_hbm = rest[0], rest[1]
            scr = rest[2:]
        else:
            out_hbm = rest[0]
            scr = rest[1:]
        srcv, dstv, onesv, dbuf, acc_sh, deg_sh = scr[:6]
        bufs = list(scr[6:6 + NBUF])
        gsems = list(scr[6 + NBUF:6 + 2 * NBUF])
        ssems = list(scr[6 + 2 * NBUF:6 + 3 * NBUF])
        dsem = scr[6 + 3 * NBUF]
        buf = bufs[0]

        c = lax.axis_index("c")
        s = lax.axis_index("s")
        base = s * rows_pt
        ptab = p_hbm.at[c]

        zero16 = jnp.zeros((16,), jnp.float32)

        def zfill(r, carry):
            for cb in range(hcb):
                buf[r, pl.ds(cb * 16, 16)] = zero16
            return carry
        lax.fori_loop(0, CH, zfill, 0)

        pltpu.sync_copy(src_hbm.at[s], srcv)
        pltpu.sync_copy(dst_hbm.at[s], dstv)
        if with_deg:
            one16 = jnp.ones((16,), jnp.float32)

            def ofill(i, carry):
                onesv[pl.ds(i * 16, 16)] = one16
                return carry
            lax.fori_loop(0, CH // 16, ofill, 0)

        def zcopy(i, carry):
            pltpu.sync_copy(buf, acc_sh.at[pl.ds(base + i * CH, CH)])
            return carry
        lax.fori_loop(0, ndrain, zcopy, 0)
        if with_deg:
            def zdeg(i, carry):
                pltpu.sync_copy(buf.at[0], deg_sh.at[pl.ds(base + i * hc, hc)])
                return carry
            lax.fori_loop(0, rows_pt // hc, zdeg, 0)
        plsc.subcore_barrier()

        for b in range(NBUF):
            pltpu.async_copy(ptab.at[srcv.at[b]], bufs[b], gsems[b])

        def group(g, carry):
            for b in range(NBUF):
                j = g * NBUF + b
                pltpu.make_async_copy(
                    ptab.at[srcv.at[j]], bufs[b], gsems[b]).wait()
                pltpu.sync_copy(bufs[b], acc_sh.at[dstv.at[j]], add=True)
                if with_deg:
                    @pl.when(c == 0)
                    def _():
                        pltpu.async_copy(onesv, deg_sh.at[dstv.at[j]],
                                         dsem, add=True)
                nj = j + NBUF

                @pl.when(nj < nchunk)
                def _():
                    pltpu.async_copy(ptab.at[srcv.at[nj]], bufs[b], gsems[b])
            return carry
        lax.fori_loop(0, nchunk // NBUF, group, 0)
        if with_deg:
            @pl.when(c == 0)
            def _():
                def ddrain(j, carry):
                    pltpu.make_async_copy(
                        onesv, deg_sh.at[dstv.at[j]], dsem).wait()
                    return carry
                lax.fori_loop(0, nchunk, ddrain, 0)
        plsc.subcore_barrier()

        if with_deg:
            pltpu.sync_copy(deg_sh.at[pl.ds(base, rows_pt)], dbuf)

            def iv(i, carry):
                d = dbuf[pl.ds(i * 16, 16)]
                dbuf[pl.ds(i * 16, 16)] = 1.0 / jnp.maximum(d, 1.0)
                return carry
            lax.fori_loop(0, rows_pt // 16, iv, 0)

            @pl.when(c == 0)
            def _():
                pltpu.sync_copy(dbuf, invd_hbm.at[pl.ds(base, rows_pt)])

        for i in range(ndrain):
            pltpu.async_copy(acc_sh.at[pl.ds(base + i * CH, CH)],
                             out_hbm.at[c].at[pl.ds(base + i * CH, CH)],
                             gsems[i % NBUF])
        for i in range(ndrain):
            pltpu.make_async_copy(
                acc_sh.at[pl.ds(base + i * CH, CH)],
                out_hbm.at[c].at[pl.ds(base + i * CH, CH)],
                gsems[i % NBUF]).wait()

    return pl.kernel(body, out_type=tuple(outs), mesh=mesh,
                     scratch_types=tuple(scratch),
                     compiler_params=pltpu.CompilerParams(
                         use_tc_tiling_on_sc=False))


def _pack_p(p, p_ref):
    rb, h = p.shape
    hc = h // NC
    p3 = p.reshape(rb // 2, 2, h)
    for c in range(NC):
        p_ref[c] = jnp.concatenate(
            [p3[:, 0, c * hc:(c + 1) * hc], p3[:, 1, c * hc:(c + 1) * hc]],
            axis=1)


def _unpack_s(sp_ref, invd_ref):
    spp = sp_ref[...]
    _, rb2, h = spp.shape
    hc = h // 2
    a = jnp.concatenate([spp[0][:, :hc], spp[1][:, :hc]], axis=1)
    b = jnp.concatenate([spp[0][:, hc:], spp[1][:, hc:]], axis=1)
    sm = jnp.stack([a, b], axis=1).reshape(rb2 * 2, h)
    return sm * invd_ref[...]


def _mm2_body(x_ref, wn_ref, ws_ref, b_ref, p_ref, q_ref):
    x = x_ref[...]
    _pack_p(jnp.dot(x, wn_ref[...], preferred_element_type=jnp.float32),
            p_ref)
    q_ref[...] = (jnp.dot(x, ws_ref[...], preferred_element_type=jnp.float32)
                  + b_ref[...])


def _layer_body(q_ref, sp_ref, invd_ref, wn_ref, ws_ref, b_ref,
                p_ref, q2_ref):
    sm = _unpack_s(sp_ref, invd_ref)
    hcur = jnp.maximum(q_ref[...] + sm, 0.0)
    _pack_p(jnp.dot(hcur, wn_ref[...], preferred_element_type=jnp.float32),
            p_ref)
    q2_ref[...] = (jnp.dot(hcur, ws_ref[...],
                           preferred_element_type=jnp.float32) + b_ref[...])


def _make_final_body(n_real, rblk):
    def _final_body(q_ref, sp_ref, invd_ref, out_ref):
        i = pl.program_id(0)
        sm = _unpack_s(sp_ref, invd_ref)
        h2 = jnp.maximum(q_ref[...] + sm, 0.0)
        rows = i * rblk + lax.broadcasted_iota(jnp.int32, (rblk, 1), 0)
        h2 = jnp.where(rows < n_real, h2, 0.0)
        part = jnp.sum(h2, axis=0, keepdims=True) * (1.0 / n_real)

        @pl.when(i == 0)
        def _():
            out_ref[...] = jnp.zeros_like(out_ref)
        out_ref[...] += part
    return _final_body


def _tc1(feat_p, wn, ws, b, rblk):
    npad, d = feat_p.shape
    h = wn.shape[1]
    return pl.pallas_call(
        _mm2_body,
        grid=(npad // rblk,),
        in_specs=[pl.BlockSpec((rblk, d), lambda i: (i, 0)),
                  pl.BlockSpec((d, h), lambda i: (0, 0)),
                  pl.BlockSpec((d, h), lambda i: (0, 0)),
                  pl.BlockSpec((1, h), lambda i: (0, 0))],
        out_specs=[pl.BlockSpec((NC, rblk // 2, h), lambda i: (0, i, 0)),
                   pl.BlockSpec((rblk, h), lambda i: (i, 0))],
        out_shape=[jax.ShapeDtypeStruct((NC, npad // 2, h), jnp.float32),
                   jax.ShapeDtypeStruct((npad, h), jnp.float32)],
    )(feat_p, wn, ws, b)


def _tc2(q, sp, invd, wn, ws, b, rblk):
    npad, h = q.shape
    return pl.pallas_call(
        _layer_body,
        grid=(npad // rblk,),
        in_specs=[pl.BlockSpec((rblk, h), lambda i: (i, 0)),
                  pl.BlockSpec((NC, rblk // 2, h), lambda i: (0, i, 0)),
                  pl.BlockSpec((rblk, 1), lambda i: (i, 0)),
                  pl.BlockSpec((h, h), lambda i: (0, 0)),
                  pl.BlockSpec((h, h), lambda i: (0, 0)),
                  pl.BlockSpec((1, h), lambda i: (0, 0))],
        out_specs=[pl.BlockSpec((NC, rblk // 2, h), lambda i: (0, i, 0)),
                   pl.BlockSpec((rblk, h), lambda i: (i, 0))],
        out_shape=[jax.ShapeDtypeStruct((NC, npad // 2, h), jnp.float32),
                   jax.ShapeDtypeStruct((npad, h), jnp.float32)],
    )(q, sp, invd, wn, ws, b)


def _tc3(q, sp, invd, n_real, rblk):
    npad, h = q.shape
    return pl.pallas_call(
        _make_final_body(n_real, rblk),
        grid=(npad // rblk,),
        in_specs=[pl.BlockSpec((rblk, h), lambda i: (i, 0)),
                  pl.BlockSpec((NC, rblk // 2, h), lambda i: (0, i, 0)),
                  pl.BlockSpec((rblk, 1), lambda i: (i, 0))],
        out_specs=pl.BlockSpec((1, h), lambda i: (0, 0)),
        out_shape=jax.ShapeDtypeStruct((1, h), jnp.float32),
    )(q, sp, invd)


def kernel(feat, edge_index, W_self1, W_neigh1, b1, W_self2, W_neigh2, b2):
    n, d = feat.shape
    e = edge_index.shape[1]
    h = W_self1.shape[0]
    rblk = 1280

    npad = -(-n // (NS * CH)) * NS * CH
    nchunk = -(-e // (NS * CH))
    nchunk = -(-nchunk // NBUF) * NBUF
    e_pad = NS * nchunk * CH
    pad = e_pad - e
    prows = npad - n

    src = edge_index[0].astype(jnp.int32)
    dst = edge_index[1].astype(jnp.int32)
    if pad:
        ar = jnp.arange(pad, dtype=jnp.int32)
        src = jnp.concatenate([src, ar % n])
        dst = jnp.concatenate([dst, n + ar % prows])
    src3 = src.reshape(NS, nchunk, CH)
    dst3 = dst.reshape(NS, nchunk, CH)

    feat_p = feat
    if npad != n:
        feat_p = jnp.concatenate(
            [feat, jnp.zeros((npad - n, d), feat.dtype)])

    wn1, ws1 = W_neigh1.T, W_self1.T
    wn2, ws2 = W_neigh2.T, W_self2.T
    b1r, b2r = b1.reshape(1, h), b2.reshape(1, h)

    hc = h // NC
    agg1 = _make_agg(npad, h, nchunk, True)
    agg2 = _make_agg(npad, h, nchunk, False)

    p1, q1 = _tc1(feat_p, wn1, ws1, b1r, rblk)
    s1, invd = agg1(p1.reshape(NC, npad, hc), src3, dst3)
    invd2 = invd.reshape(npad, 1)
    p2, q2 = _tc2(q1, s1.reshape(NC, npad // 2, h), invd2, wn2, ws2, b2r,
                  rblk)
    (s2,) = agg2(p2.reshape(NC, npad, hc), src3, dst3)
    out = _tc3(q2, s2.reshape(NC, npad // 2, h), invd2, n, rblk)
    return out.reshape(h)


# speedup vs baseline: 1.3181x; 1.0102x over previous
import functools

import jax
import jax.numpy as jnp
from jax import lax
from jax.experimental import pallas as pl
from jax.experimental.pallas import tpu as pltpu
from jax.experimental.pallas import tpu_sc as plsc

NC = 2
NS = 16
CH = 128
NBUF = 5


@functools.lru_cache(maxsize=None)
def _make_agg(n_pad, h, nchunk, with_deg):
    hc = h // NC
    rows_pt = n_pad // NS
    ndrain = rows_pt // CH
    hcb = hc // 16

    mesh = plsc.VectorSubcoreMesh(
        core_axis_name="c", subcore_axis_name="s",
        num_cores=NC, num_subcores=NS)

    outs = [jax.ShapeDtypeStruct((NC, n_pad, hc), jnp.float32)]
    if with_deg:
        outs.append(jax.ShapeDtypeStruct((n_pad,), jnp.float32))

    scratch = [
        pltpu.VMEM((nchunk, CH), jnp.int32),
        pltpu.VMEM((nchunk, CH), jnp.int32),
        pltpu.VMEM((CH,), jnp.float32),
        pltpu.VMEM((rows_pt,), jnp.float32),
        pltpu.VMEM_SHARED((n_pad, hc), jnp.float32),
        pltpu.VMEM_SHARED((n_pad,), jnp.float32),
    ] + [pltpu.VMEM((CH, hc), jnp.float32) for _ in range(NBUF)] \
      + [pltpu.SemaphoreType.DMA for _ in range(NBUF + 1)]

    def body(p_hbm, src_hbm, dst_hbm, *rest):
        if with_deg:
            out_hbm, invd_hbm = rest[0], rest[1]
            scr = rest[2:]
        else:
            out_hbm = rest[0]
            scr = rest[1:]
        srcv, dstv, onesv, dbuf, acc_sh, deg_sh = scr[:6]
        bufs = list(scr[6:6 + NBUF])
        gsems = list(scr[6 + NBUF:6 + 2 * NBUF])
        dsem = scr[6 + 2 * NBUF]
        buf = bufs[0]

        c = lax.axis_index("c")
        s = lax.axis_index("s")
        base = s * rows_pt
        ptab = p_hbm.at[c]

        pltpu.sync_copy(src_hbm.at[s], srcv)
        pltpu.sync_copy(dst_hbm.at[s], dstv)
        for b in range(1, NBUF):
            pltpu.async_copy(ptab.at[srcv.at[b]], bufs[b], gsems[b])

        zero16 = jnp.zeros((16,), jnp.float32)

        def zfill(r, carry):
            for cb in range(hcb):
                buf[r, pl.ds(cb * 16, 16)] = zero16
            return carry
        lax.fori_loop(0, CH, zfill, 0)

        if with_deg:
            one16 = jnp.ones((16,), jnp.float32)

            def ofill(i, carry):
                onesv[pl.ds(i * 16, 16)] = one16
                return carry
            lax.fori_loop(0, CH // 16, ofill, 0)

        def zcopy(i, carry):
            pltpu.sync_copy(buf, acc_sh.at[pl.ds(base + i * CH, CH)])
            return carry
        lax.fori_loop(0, ndrain, zcopy, 0)
        if with_deg:
            def zdeg(i, carry):
                pltpu.sync_copy(buf.at[0], deg_sh.at[pl.ds(base + i * hc, hc)])
                return carry
            lax.fori_loop(0, rows_pt // hc, zdeg, 0)
        plsc.subcore_barrier()

        pltpu.async_copy(ptab.at[srcv.at[0]], bufs[0], gsems[0])

        def group(g, carry):
            for b in range(NBUF):
                j = g * NBUF + b
                pltpu.make_async_copy(
                    ptab.at[srcv.at[j]], bufs[b], gsems[b]).wait()
                pltpu.sync_copy(bufs[b], acc_sh.at[dstv.at[j]], add=True)
                if with_deg:
                    @pl.when(c == 0)
                    def _():
                        pltpu.async_copy(onesv, deg_sh.at[dstv.at[j]],
                                         dsem, add=True)
                nj = j + NBUF

                @pl.when(nj < nchunk)
                def _():
                    pltpu.async_copy(ptab.at[srcv.at[nj]], bufs[b], gsems[b])
            return carry
        lax.fori_loop(0, nchunk // NBUF, group, 0)
        if with_deg:
            @pl.when(c == 0)
            def _():
                def ddrain(j, carry):
                    pltpu.make_async_copy(
                        onesv, deg_sh.at[dstv.at[j]], dsem).wait()
                    return carry
                lax.fori_loop(0, nchunk, ddrain, 0)
        plsc.subcore_barrier()

        if with_deg:
            pltpu.sync_copy(deg_sh.at[pl.ds(base, rows_pt)], dbuf)

            def iv(i, carry):
                d = dbuf[pl.ds(i * 16, 16)]
                dbuf[pl.ds(i * 16, 16)] = 1.0 / jnp.maximum(d, 1.0)
                return carry
            lax.fori_loop(0, rows_pt // 16, iv, 0)

            @pl.when(c == 0)
            def _():
                pltpu.sync_copy(dbuf, invd_hbm.at[pl.ds(base, rows_pt)])

        for i in range(ndrain):
            pltpu.async_copy(acc_sh.at[pl.ds(base + i * CH, CH)],
                             out_hbm.at[c].at[pl.ds(base + i * CH, CH)],
                             gsems[i % NBUF])
        for i in range(ndrain):
            pltpu.make_async_copy(
                acc_sh.at[pl.ds(base + i * CH, CH)],
                out_hbm.at[c].at[pl.ds(base + i * CH, CH)],
                gsems[i % NBUF]).wait()

    return pl.kernel(body, out_type=tuple(outs), mesh=mesh,
                     scratch_types=tuple(scratch),
                     compiler_params=pltpu.CompilerParams(
                         use_tc_tiling_on_sc=False))


def _pack_p(p, p_ref):
    rb, h = p.shape
    hc = h // NC
    p3 = p.reshape(rb // 2, 2, h)
    for c in range(NC):
        p_ref[c] = jnp.concatenate(
            [p3[:, 0, c * hc:(c + 1) * hc], p3[:, 1, c * hc:(c + 1) * hc]],
            axis=1)


def _unpack_s(sp_ref, invd_ref):
    spp = sp_ref[...]
    _, rb2, h = spp.shape
    hc = h // 2
    a = jnp.concatenate([spp[0][:, :hc], spp[1][:, :hc]], axis=1)
    b = jnp.concatenate([spp[0][:, hc:], spp[1][:, hc:]], axis=1)
    sm = jnp.stack([a, b], axis=1).reshape(rb2 * 2, h)
    return sm * invd_ref[...]


def _mm2_body(x_ref, wn_ref, ws_ref, b_ref, p_ref, q_ref):
    x = x_ref[...]
    _pack_p(jnp.dot(x, wn_ref[...], preferred_element_type=jnp.float32),
            p_ref)
    q_ref[...] = (jnp.dot(x, ws_ref[...], preferred_element_type=jnp.float32)
                  + b_ref[...])


def _layer_body(q_ref, sp_ref, invd_ref, wn_ref, ws_ref, b_ref,
                p_ref, q2_ref):
    sm = _unpack_s(sp_ref, invd_ref)
    hcur = jnp.maximum(q_ref[...] + sm, 0.0)
    _pack_p(jnp.dot(hcur, wn_ref[...], preferred_element_type=jnp.float32),
            p_ref)
    q2_ref[...] = (jnp.dot(hcur, ws_ref[...],
                           preferred_element_type=jnp.float32) + b_ref[...])


def _make_final_body(n_real, rblk):
    def _final_body(q_ref, sp_ref, invd_ref, out_ref):
        i = pl.program_id(0)
        sm = _unpack_s(sp_ref, invd_ref)
        h2 = jnp.maximum(q_ref[...] + sm, 0.0)
        rows = i * rblk + lax.broadcasted_iota(jnp.int32, (rblk, 1), 0)
        h2 = jnp.where(rows < n_real, h2, 0.0)
        part = jnp.sum(h2, axis=0, keepdims=True) * (1.0 / n_real)

        @pl.when(i == 0)
        def _():
            out_ref[...] = jnp.zeros_like(out_ref)
        out_ref[...] += part
    return _final_body


def _tc1(feat_p, wn, ws, b, rblk):
    npad, d = feat_p.shape
    h = wn.shape[1]
    return pl.pallas_call(
        _mm2_body,
        grid=(npad // rblk,),
        in_specs=[pl.BlockSpec((rblk, d), lambda i: (i, 0)),
                  pl.BlockSpec((d, h), lambda i: (0, 0)),
                  pl.BlockSpec((d, h), lambda i: (0, 0)),
                  pl.BlockSpec((1, h), lambda i: (0, 0))],
        out_specs=[pl.BlockSpec((NC, rblk // 2, h), lambda i: (0, i, 0)),
                   pl.BlockSpec((rblk, h), lambda i: (i, 0))],
        out_shape=[jax.ShapeDtypeStruct((NC, npad // 2, h), jnp.float32),
                   jax.ShapeDtypeStruct((npad, h), jnp.float32)],
    )(feat_p, wn, ws, b)


def _tc2(q, sp, invd, wn, ws, b, rblk):
    npad, h = q.shape
    return pl.pallas_call(
        _layer_body,
        grid=(npad // rblk,),
        in_specs=[pl.BlockSpec((rblk, h), lambda i: (i, 0)),
                  pl.BlockSpec((NC, rblk // 2, h), lambda i: (0, i, 0)),
                  pl.BlockSpec((rblk, 1), lambda i: (i, 0)),
                  pl.BlockSpec((h, h), lambda i: (0, 0)),
                  pl.BlockSpec((h, h), lambda i: (0, 0)),
                  pl.BlockSpec((1, h), lambda i: (0, 0))],
        out_specs=[pl.BlockSpec((NC, rblk // 2, h), lambda i: (0, i, 0)),
                   pl.BlockSpec((rblk, h), lambda i: (i, 0))],
        out_shape=[jax.ShapeDtypeStruct((NC, npad // 2, h), jnp.float32),
                   jax.ShapeDtypeStruct((npad, h), jnp.float32)],
    )(q, sp, invd, wn, ws, b)


def _tc3(q, sp, invd, n_real, rblk):
    npad, h = q.shape
    return pl.pallas_call(
        _make_final_body(n_real, rblk),
        grid=(npad // rblk,),
        in_specs=[pl.BlockSpec((rblk, h), lambda i: (i, 0)),
                  pl.BlockSpec((NC, rblk // 2, h), lambda i: (0, i, 0)),
                  pl.BlockSpec((rblk, 1), lambda i: (i, 0))],
        out_specs=pl.BlockSpec((1, h), lambda i: (0, 0)),
        out_shape=jax.ShapeDtypeStruct((1, h), jnp.float32),
    )(q, sp, invd)


def kernel(feat, edge_index, W_self1, W_neigh1, b1, W_self2, W_neigh2, b2):
    n, d = feat.shape
    e = edge_index.shape[1]
    h = W_self1.shape[0]
    rblk = 1280

    npad = -(-n // (NS * CH)) * NS * CH
    nchunk = -(-e // (NS * CH))
    nchunk = -(-nchunk // NBUF) * NBUF
    e_pad = NS * nchunk * CH
    pad = e_pad - e
    prows = npad - n

    src = edge_index[0].astype(jnp.int32)
    dst = edge_index[1].astype(jnp.int32)
    if pad:
        ar = jnp.arange(pad, dtype=jnp.int32)
        src = jnp.concatenate([src, ar % n])
        dst = jnp.concatenate([dst, n + ar % prows])
    src3 = src.reshape(NS, nchunk, CH)
    dst3 = dst.reshape(NS, nchunk, CH)

    feat_p = feat
    if npad != n:
        feat_p = jnp.concatenate(
            [feat, jnp.zeros((npad - n, d), feat.dtype)])

    wn1, ws1 = W_neigh1.T, W_self1.T
    wn2, ws2 = W_neigh2.T, W_self2.T
    b1r, b2r = b1.reshape(1, h), b2.reshape(1, h)

    hc = h // NC
    agg1 = _make_agg(npad, h, nchunk, True)
    agg2 = _make_agg(npad, h, nchunk, False)

    p1, q1 = _tc1(feat_p, wn1, ws1, b1r, rblk)
    s1, invd = agg1(p1.reshape(NC, npad, hc), src3, dst3)
    invd2 = invd.reshape(npad, 1)
    p2, q2 = _tc2(q1, s1.reshape(NC, npad // 2, h), invd2, wn2, ws2, b2r,
                  rblk)
    (s2,) = agg2(p2.reshape(NC, npad, hc), src3, dst3)
    out = _tc3(q2, s2.reshape(NC, npad // 2, h), invd2, n, rblk)
    return out.reshape(h)


# speedup vs baseline: 1.3375x; 1.0147x over previous
import functools

import jax
import jax.numpy as jnp
from jax import lax
from jax.experimental import pallas as pl
from jax.experimental.pallas import tpu as pltpu
from jax.experimental.pallas import tpu_sc as plsc

NC = 2
NS = 16
CH = 128
NBUF = 5


@functools.lru_cache(maxsize=None)
def _make_agg(n_pad, h, nchunk, with_deg):
    hc = h // NC
    rows_pt = n_pad // NS
    ndrain = rows_pt // CH
    hcb = hc // 16

    mesh = plsc.VectorSubcoreMesh(
        core_axis_name="c", subcore_axis_name="s",
        num_cores=NC, num_subcores=NS)

    outs = [jax.ShapeDtypeStruct((NC, n_pad, hc), jnp.float32)]
    if with_deg:
        outs.append(jax.ShapeDtypeStruct((n_pad,), jnp.float32))

    scratch = [
        pltpu.VMEM((nchunk, CH), jnp.int32),
        pltpu.VMEM((nchunk, CH), jnp.int32),
        pltpu.VMEM((CH,), jnp.float32),
        pltpu.VMEM((rows_pt,), jnp.float32),
        pltpu.VMEM_SHARED((n_pad, hc), jnp.float32),
        pltpu.VMEM_SHARED((n_pad,), jnp.float32),
    ] + [pltpu.VMEM((CH, hc), jnp.float32) for _ in range(NBUF)] \
      + [pltpu.SemaphoreType.DMA for _ in range(NBUF + 1)]

    def body(p_hbm, src_hbm, dst_hbm, *rest):
        if with_deg:
            out_hbm, invd_hbm = rest[0], rest[1]
            scr = rest[2:]
        else:
            out_hbm = rest[0]
            scr = rest[1:]
        srcv, dstv, onesv, dbuf, acc_sh, deg_sh = scr[:6]
        bufs = list(scr[6:6 + NBUF])
        gsems = list(scr[6 + NBUF:6 + 2 * NBUF])
        dsem = scr[6 + 2 * NBUF]
        buf = bufs[0]

        c = lax.axis_index("c")
        s = lax.axis_index("s")
        base = s * rows_pt
        ptab = p_hbm.at[c]

        pltpu.sync_copy(src_hbm.at[s], srcv)
        pltpu.sync_copy(dst_hbm.at[s], dstv)
        for b in range(1, NBUF):
            pltpu.async_copy(ptab.at[srcv.at[b]], bufs[b], gsems[b])

        zero16 = jnp.zeros((16,), jnp.float32)

        def zfill(r, carry):
            for cb in range(hcb):
                buf[r, pl.ds(cb * 16, 16)] = zero16
            return carry
        lax.fori_loop(0, CH, zfill, 0)

        if with_deg:
            one16 = jnp.ones((16,), jnp.float32)

            def ofill(i, carry):
                onesv[pl.ds(i * 16, 16)] = one16
                return carry
            lax.fori_loop(0, CH // 16, ofill, 0)

        def zcopy(i, carry):
            pltpu.sync_copy(buf, acc_sh.at[pl.ds(base + i * CH, CH)])
            return carry
        lax.fori_loop(0, ndrain, zcopy, 0)
        if with_deg:
            def zdeg(i, carry):
                pltpu.sync_copy(buf.at[0], deg_sh.at[pl.ds(base + i * hc, hc)])
                return carry
            lax.fori_loop(0, rows_pt // hc, zdeg, 0)
        plsc.subcore_barrier()

        pltpu.async_copy(ptab.at[srcv.at[0]], bufs[0], gsems[0])

        def group(g, carry):
            for b in range(NBUF):
                j = g * NBUF + b
                pltpu.make_async_copy(
                    ptab.at[srcv.at[j]], bufs[b], gsems[b]).wait()
                pltpu.sync_copy(bufs[b], acc_sh.at[dstv.at[j]], add=True)
                if with_deg:
                    @pl.when(c == 0)
                    def _():
                        pltpu.async_copy(onesv, deg_sh.at[dstv.at[j]],
                                         dsem, add=True)
                nj = j + NBUF

                @pl.when(nj < nchunk)
                def _():
                    pltpu.async_copy(ptab.at[srcv.at[nj]], bufs[b], gsems[b])
            return carry
        lax.fori_loop(0, nchunk // NBUF, group, 0)
        if with_deg:
            @pl.when(c == 0)
            def _():
                def ddrain(j, carry):
                    pltpu.make_async_copy(
                        onesv, deg_sh.at[dstv.at[j]], dsem).wait()
                    return carry
                lax.fori_loop(0, nchunk, ddrain, 0)
        plsc.subcore_barrier()

        if with_deg:
            pltpu.sync_copy(deg_sh.at[pl.ds(base, rows_pt)], dbuf)

            def iv(i, carry):
                d = dbuf[pl.ds(i * 16, 16)]
                dbuf[pl.ds(i * 16, 16)] = 1.0 / jnp.maximum(d, 1.0)
                return carry
            lax.fori_loop(0, rows_pt // 16, iv, 0)

            @pl.when(c == 0)
            def _():
                pltpu.sync_copy(dbuf, invd_hbm.at[pl.ds(base, rows_pt)])

        for i in range(ndrain):
            pltpu.async_copy(acc_sh.at[pl.ds(base + i * CH, CH)],
                             out_hbm.at[c].at[pl.ds(base + i * CH, CH)],
                             gsems[i % NBUF])
        for i in range(ndrain):
            pltpu.make_async_copy(
                acc_sh.at[pl.ds(base + i * CH, CH)],
                out_hbm.at[c].at[pl.ds(base + i * CH, CH)],
                gsems[i % NBUF]).wait()

    return pl.kernel(body, out_type=tuple(outs), mesh=mesh,
                     scratch_types=tuple(scratch),
                     compiler_params=pltpu.CompilerParams(
                         use_tc_tiling_on_sc=False))


def _pack_p(p, p_ref):
    rb, h = p.shape
    hc = h // NC
    p3 = p.reshape(rb // 2, 2, h)
    for c in range(NC):
        p_ref[c] = jnp.concatenate(
            [p3[:, 0, c * hc:(c + 1) * hc], p3[:, 1, c * hc:(c + 1) * hc]],
            axis=1)


def _unpack_s(sp_ref, invd_ref):
    spp = sp_ref[...]
    _, rb2, h = spp.shape
    hc = h // 2
    a = jnp.concatenate([spp[0][:, :hc], spp[1][:, :hc]], axis=1)
    b = jnp.concatenate([spp[0][:, hc:], spp[1][:, hc:]], axis=1)
    sm = jnp.stack([a, b], axis=1).reshape(rb2 * 2, h)
    return sm * invd_ref[...]


def _mm2_body(x_ref, wn_ref, ws_ref, b_ref, p_ref, q_ref):
    x = x_ref[...]
    _pack_p(jnp.dot(x, wn_ref[...], preferred_element_type=jnp.float32),
            p_ref)
    q_ref[...] = (jnp.dot(x, ws_ref[...], preferred_element_type=jnp.float32)
                  + b_ref[...])


def _layer_body(q_ref, sp_ref, invd_ref, wn_ref, ws_ref, b_ref,
                p_ref, q2_ref):
    sm = _unpack_s(sp_ref, invd_ref)
    hcur = jnp.maximum(q_ref[...] + sm, 0.0)
    _pack_p(jnp.dot(hcur, wn_ref[...], preferred_element_type=jnp.float32),
            p_ref)
    q2_ref[...] = (jnp.dot(hcur, ws_ref[...],
                           preferred_element_type=jnp.float32) + b_ref[...])


def _make_final_body(n_real, rblk):
    def _final_body(q_ref, sp_ref, invd_ref, out_ref):
        i = pl.program_id(0)
        sm = _unpack_s(sp_ref, invd_ref)
        h2 = jnp.maximum(q_ref[...] + sm, 0.0)
        rows = i * rblk + lax.broadcasted_iota(jnp.int32, (rblk, 1), 0)
        h2 = jnp.where(rows < n_real, h2, 0.0)
        part = jnp.sum(h2, axis=0, keepdims=True) * (1.0 / n_real)

        @pl.when(i == 0)
        def _():
            out_ref[...] = jnp.zeros_like(out_ref)
        out_ref[...] += part
    return _final_body


def _tc1(feat, wn, ws, b, rblk, npad):
    n, d = feat.shape
    h = wn.shape[1]
    return pl.pallas_call(
        _mm2_body,
        grid=(n // rblk,),
        in_specs=[pl.BlockSpec((rblk, d), lambda i: (i, 0)),
                  pl.BlockSpec((d, h), lambda i: (0, 0)),
                  pl.BlockSpec((d, h), lambda i: (0, 0)),
                  pl.BlockSpec((1, h), lambda i: (0, 0))],
        out_specs=[pl.BlockSpec((NC, rblk // 2, h), lambda i: (0, i, 0)),
                   pl.BlockSpec((rblk, h), lambda i: (i, 0))],
        out_shape=[jax.ShapeDtypeStruct((NC, npad // 2, h), jnp.float32),
                   jax.ShapeDtypeStruct((npad, h), jnp.float32)],
    )(feat, wn, ws, b)


def _tc2(q, sp, invd, wn, ws, b, rblk):
    npad, h = q.shape
    return pl.pallas_call(
        _layer_body,
        grid=(npad // rblk,),
        in_specs=[pl.BlockSpec((rblk, h), lambda i: (i, 0)),
                  pl.BlockSpec((NC, rblk // 2, h), lambda i: (0, i, 0)),
                  pl.BlockSpec((rblk, 1), lambda i: (i, 0)),
                  pl.BlockSpec((h, h), lambda i: (0, 0)),
                  pl.BlockSpec((h, h), lambda i: (0, 0)),
                  pl.BlockSpec((1, h), lambda i: (0, 0))],
        out_specs=[pl.BlockSpec((NC, rblk // 2, h), lambda i: (0, i, 0)),
                   pl.BlockSpec((rblk, h), lambda i: (i, 0))],
        out_shape=[jax.ShapeDtypeStruct((NC, npad // 2, h), jnp.float32),
                   jax.ShapeDtypeStruct((npad, h), jnp.float32)],
    )(q, sp, invd, wn, ws, b)


def _tc3(q, sp, invd, n_real, rblk):
    npad, h = q.shape
    return pl.pallas_call(
        _make_final_body(n_real, rblk),
        grid=(npad // rblk,),
        in_specs=[pl.BlockSpec((rblk, h), lambda i: (i, 0)),
                  pl.BlockSpec((NC, rblk // 2, h), lambda i: (0, i, 0)),
                  pl.BlockSpec((rblk, 1), lambda i: (i, 0))],
        out_specs=pl.BlockSpec((1, h), lambda i: (0, 0)),
        out_shape=jax.ShapeDtypeStruct((1, h), jnp.float32),
    )(q, sp, invd)


def kernel(feat, edge_index, W_self1, W_neigh1, b1, W_self2, W_neigh2, b2):
    n, d = feat.shape
    e = edge_index.shape[1]
    h = W_self1.shape[0]
    rblk = 1280

    npad = -(-n // (NS * CH)) * NS * CH
    nchunk = -(-e // (NS * CH))
    nchunk = -(-nchunk // NBUF) * NBUF
    e_pad = NS * nchunk * CH
    pad = e_pad - e
    prows = npad - n

    src = edge_index[0].astype(jnp.int32)
    dst = edge_index[1].astype(jnp.int32)
    if pad:
        ar = jnp.arange(pad, dtype=jnp.int32)
        src = jnp.concatenate([src, ar % n])
        dst = jnp.concatenate([dst, n + ar % prows])
    src3 = src.reshape(NS, nchunk, CH)
    dst3 = dst.reshape(NS, nchunk, CH)

    wn1, ws1 = W_neigh1.T, W_self1.T
    wn2, ws2 = W_neigh2.T, W_self2.T
    b1r, b2r = b1.reshape(1, h), b2.reshape(1, h)

    hc = h // NC
    agg1 = _make_agg(npad, h, nchunk, True)
    agg2 = _make_agg(npad, h, nchunk, False)

    rblk1 = 2000
    p1, q1 = _tc1(feat, wn1, ws1, b1r, rblk1, npad)
    s1, invd = agg1(p1.reshape(NC, npad, hc), src3, dst3)
    invd2 = invd.reshape(npad, 1)
    p2, q2 = _tc2(q1, s1.reshape(NC, npad // 2, h), invd2, wn2, ws2, b2r,
                  rblk)
    (s2,) = agg2(p2.reshape(NC, npad, hc), src3, dst3)
    out = _tc3(q2, s2.reshape(NC, npad // 2, h), invd2, n, rblk)
    return out.reshape(h)
